# Initial kernel scaffold; baseline (speedup 1.0000x reference)
import jax
import jax.numpy as jnp
from jax.experimental import pallas as pl


def kernel(Corpus_, batch_inputs, entity_embeddings, relation_embed, edge_list, edge_type, edge_embed, edge_list_nhop, edge_type_nhop, a_in, a2_in, a_out, a2_out, W, a_final, a2_final, W_emb, a_emb):
    raise NotImplementedError("write your pallas kernel here")


# speedup vs baseline: 2.9959x; 2.9959x over previous
import functools

import jax
import jax.numpy as jnp
from jax import lax
from jax.experimental import pallas as pl
from jax.experimental.pallas import tpu as pltpu
from jax.experimental.pallas import tpu_sc as plsc

ALPHA = 0.2
N_NODES = 10000
NC = 2
NS = 16
L = 16
NW = NC * NS
C = 64
CF = 32
WROW = 128
NACC = 10240
PADROW = 10008
RPT = NACC // NS
RB = 64
_CP = pltpu.CompilerParams(needs_layout_passes=False)
_MESH = dict(core_axis_name="c", subcore_axis_name="s")


def _zero_acc(sid, zsrc, acc_sh):
  zero = jnp.zeros((L,), jnp.float32)
  for r in range(16):
    for j in range(WROW // L):
      zsrc[r, pl.ds(j * L, L)] = zero

  def zloop(i, carry):
    pltpu.sync_copy(zsrc, acc_sh.at[pl.ds(sid * RPT + i * 16, 16)])
    return carry
  lax.fori_loop(0, RPT // 16, zloop, 0)


def _write_acc(cid, sid, acc_sh, obuf, out_hbm, rb):
  for b in range(RPT // rb):
    pltpu.sync_copy(acc_sh.at[pl.ds(sid * RPT + b * rb, rb)], obuf)
    pltpu.sync_copy(obuf, out_hbm.at[cid, pl.ds(sid * RPT + b * rb, rb)])


def _attn_e(tab1_v, tab2_v, sv, ov, srg):
  s1g = plsc.load_gather(tab1_v, [sv])
  s2g = plsc.load_gather(tab2_v, [ov])
  p = s1g + s2g + srg
  return jnp.exp(jnp.where(p > 0.0, -p, -ALPHA * p))


def _layer1_pass(nch):
  @functools.partial(
      pl.kernel,
      mesh=plsc.VectorSubcoreMesh(**_MESH),
      out_type=jax.ShapeDtypeStruct((NC, NACC, WROW), jnp.float32),
      compiler_params=_CP,
      scratch_types=[
          pltpu.VMEM((C,), jnp.int32),
          pltpu.VMEM((C,), jnp.int32),
          pltpu.VMEM((C,), jnp.float32),
          pltpu.VMEM((C, 16), jnp.float32),
          pltpu.VMEM((C, WROW), jnp.float32),
          pltpu.VMEM((C, WROW), jnp.float32),
          pltpu.VMEM((NACC,), jnp.float32),
          pltpu.VMEM((NACC,), jnp.float32),
          pltpu.VMEM_SHARED((NACC, WROW), jnp.float32),
      ],
  )
  def kern(seg_hbm, oth_hbm, sr_hbm, rel_hbm, v_hbm, s1_hbm, s2_hbm,
           out_hbm, seg_v, oth_v, sr_v, rel_v, rows_v, scat_v,
           tab1_v, tab2_v, acc_sh):
    cid = lax.axis_index("c")
    sid = lax.axis_index("s")
    wid = cid * NS + sid

    pltpu.sync_copy(s1_hbm, tab1_v)
    pltpu.sync_copy(s2_hbm, tab2_v)
    _zero_acc(sid, rows_v.at[pl.ds(0, 16)], acc_sh)
    zero = jnp.zeros((L,), jnp.float32)
    for r in range(C):
      scat_v[r, pl.ds(96, L)] = zero
      scat_v[r, pl.ds(112, L)] = zero
    plsc.subcore_barrier()

    lane = lax.iota(jnp.int32, 16)
    ebase = wid * nch * C

    def chunk(c, carry):
      eoff = ebase + c * C
      pltpu.sync_copy(seg_hbm.at[pl.ds(eoff, C)], seg_v)
      pltpu.sync_copy(oth_hbm.at[pl.ds(eoff, C)], oth_v)
      pltpu.sync_copy(sr_hbm.at[pl.ds(eoff, C)], sr_v)
      pltpu.sync_copy(rel_hbm.at[pl.ds(eoff, C)], rel_v)
      pltpu.sync_copy(v_hbm.at[oth_v], rows_v)

      for g in range(C // L):
        sv = seg_v[pl.ds(g * L, L)]
        ov = oth_v[pl.ds(g * L, L)]
        ee = _attn_e(tab1_v, tab2_v, sv, ov, sr_v[pl.ds(g * L, L)])
        for k in range(L):
          r = g * L + k
          eb = ee.at[jnp.full((L,), k, jnp.int32)].get(
              mode="promise_in_bounds")
          for j in range(4):
            scat_v[r, pl.ds(j * L, L)] = rows_v[r, pl.ds(j * L, L)] * eb
          scat_v[r, pl.ds(64, L)] = rel_v[r, :] * eb
          scat_v[r, pl.ds(80, L)] = jnp.where(lane == 0, eb, 0.0)

      pltpu.sync_copy(scat_v, acc_sh.at[seg_v], add=True)
      return carry
    lax.fori_loop(0, nch, chunk, 0)
    plsc.subcore_barrier()
    _write_acc(cid, sid, acc_sh, scat_v, out_hbm, C)

  return kern


def _final_pass(nch):
  @functools.partial(
      pl.kernel,
      mesh=plsc.VectorSubcoreMesh(**_MESH),
      out_type=(jax.ShapeDtypeStruct((NC, NACC, WROW), jnp.float32),
                jax.ShapeDtypeStruct((NW, NACC), jnp.float32)),
      compiler_params=_CP,
      scratch_types=[
          pltpu.VMEM((CF,), jnp.int32),
          pltpu.VMEM((CF,), jnp.int32),
          pltpu.VMEM((CF,), jnp.float32),
          pltpu.VMEM((CF, WROW), jnp.float32),
          pltpu.VMEM((CF, WROW), jnp.float32),
          pltpu.VMEM((NACC,), jnp.float32),
          pltpu.VMEM((NACC,), jnp.float32),
          pltpu.VMEM((NACC,), jnp.float32),
          pltpu.VMEM_SHARED((NACC, WROW), jnp.float32),
      ],
  )
  def kern(seg_hbm, oth_hbm, sr_hbm, relm_hbm, v_hbm, s1_hbm, s2_hbm,
           out_hbm, out2_hbm, seg_v, oth_v, sr_v, relm_v, scat_v,
           tab1_v, tab2_v, rs_priv, acc_sh):
    cid = lax.axis_index("c")
    sid = lax.axis_index("s")
    wid = cid * NS + sid

    pltpu.sync_copy(s1_hbm, tab1_v)
    pltpu.sync_copy(s2_hbm, tab2_v)
    _zero_acc(sid, relm_v.at[pl.ds(0, 16)], acc_sh)
    zero = jnp.zeros((L,), jnp.float32)
    for i in range(NACC // L):
      rs_priv[pl.ds(i * L, L)] = zero
    plsc.subcore_barrier()

    nchf = nch * (C // CF)
    ebase = wid * nchf * CF

    def chunk(c, carry):
      eoff = ebase + c * CF
      pltpu.sync_copy(seg_hbm.at[pl.ds(eoff, CF)], seg_v)
      pltpu.sync_copy(oth_hbm.at[pl.ds(eoff, CF)], oth_v)
      pltpu.sync_copy(sr_hbm.at[pl.ds(eoff, CF)], sr_v)
      pltpu.sync_copy(relm_hbm.at[pl.ds(eoff, CF)], relm_v)
      pltpu.sync_copy(v_hbm.at[oth_v], scat_v)

      for g in range(CF // L):
        sv = seg_v[pl.ds(g * L, L)]
        ov = oth_v[pl.ds(g * L, L)]
        ee = _attn_e(tab1_v, tab2_v, sv, ov, sr_v[pl.ds(g * L, L)])
        plsc.addupdate_scatter(rs_priv, [sv], ee)
        for k in range(L):
          r = g * L + k
          eb = ee.at[jnp.full((L,), k, jnp.int32)].get(
              mode="promise_in_bounds")
          for j in range(WROW // L):
            cs = pl.ds(j * L, L)
            scat_v[r, cs] = (scat_v[r, cs] + relm_v[r, cs]) * eb

      pltpu.sync_copy(scat_v, acc_sh.at[seg_v], add=True)
      return carry
    lax.fori_loop(0, nchf, chunk, 0)
    plsc.subcore_barrier()
    _write_acc(cid, sid, acc_sh, relm_v, out_hbm, CF)
    pltpu.sync_copy(rs_priv, out2_hbm.at[wid])

  return kern


def _pad_edges(seg, oth, ep):
  e = seg.shape[0]
  segp = jnp.full((ep,), PADROW, jnp.int32).at[:e].set(seg.astype(jnp.int32))
  othp = jnp.zeros((ep,), jnp.int32).at[:e].set(oth.astype(jnp.int32))
  return segp, othp


def _pad_rows(a, ep):
  return jnp.zeros((ep, a.shape[1]), jnp.float32).at[:a.shape[0]].set(a)


def _pad_tab(t):
  n = min(N_NODES, NACC)
  return jnp.zeros((NACC,), jnp.float32).at[:n].set(t[:n])


def kernel(Corpus_, batch_inputs, entity_embeddings, relation_embed,
           edge_list, edge_type, edge_embed, edge_list_nhop, edge_type_nhop,
           a_in, a2_in, a_out, a2_out, W, a_final, a2_final, W_emb, a_emb):
  x = entity_embeddings
  R = relation_embed
  t0 = edge_type_nhop[:, 0]
  t1 = edge_type_nhop[:, 1]

  e_total = edge_list.shape[1] + edge_list_nhop.shape[1]
  nch = -(-e_total // (NW * C))
  ep = NW * C * nch

  seg_in = jnp.concatenate([edge_list[0], edge_list_nhop[0]])
  oth_in = jnp.concatenate([edge_list[1], edge_list_nhop[1]])
  rel_l1 = _pad_rows(jnp.concatenate([edge_embed, R[t0] + R[t1]], axis=0), ep)
  rel_fin = _pad_rows(jnp.concatenate([R[edge_type], R[t0] + R[t1]], axis=0),
                      ep)

  seg_i, oth_i = _pad_edges(seg_in, oth_in, ep)
  seg_o, oth_o = _pad_edges(oth_in, seg_in, ep)

  l1_pass = _layer1_pass(nch)
  fin_pass = _final_pass(nch)

  def run_layer1(a, a2, seg, oth):
    hs = []
    for h in range(2):
      a_s, a_d, a_r = a[h, :, :128], a[h, :, 128:256], a[h, :, 256:]
      v = a2[h, 0, :]
      s1 = _pad_tab(x @ (v @ a_s))
      s2 = _pad_tab(x @ (v @ a_d))
      sr = rel_l1 @ (v @ a_r)
      v_tab = jnp.pad(x @ a_d.T, ((0, 0), (0, 64)))
      out = l1_pass(seg, oth, sr, rel_l1, v_tab, s1, s2)
      acc = (out[0] + out[1])[:N_NODES]
      rs = acc[:, 80]
      rs_cl = jnp.where(rs == 0.0, 1e-12, rs)
      num = (rs[:, None] * (x @ a_s.T) + acc[:, :64]
             + acc[:, 64:80] @ a_r.T)
      hs.append(num / rs_cl[:, None])
    return jax.nn.elu(jnp.concatenate(hs, axis=1))

  xs1 = run_layer1(a_in, a2_in, seg_i, oth_i)
  xs2 = run_layer1(a_out, a2_out, seg_o, oth_o)

  out_rel = R @ W

  a_sf, a_df, a_rf = a_final[:, :128], a_final[:, 128:256], a_final[:, 256:]
  vf = a2_final[0, :]
  wc = W @ a_rf.T
  relm_f = rel_fin @ wc
  sr_f = rel_fin @ (W @ (vf @ a_rf))

  def run_final(xd, seg, oth):
    s1 = _pad_tab(xd @ (vf @ a_sf))
    s2 = _pad_tab(xd @ (vf @ a_df))
    v_tab = xd @ a_df.T
    out, rs_out = fin_pass(seg, oth, sr_f, relm_f, v_tab, s1, s2)
    acc = (out[0] + out[1])[:N_NODES]
    rs = jnp.sum(rs_out, axis=0)[:N_NODES]
    rs_cl = jnp.where(rs == 0.0, 1e-12, rs)
    num = rs[:, None] * (xd @ a_sf.T) + acc
    return jax.nn.elu(num / rs_cl[:, None])

  h1 = run_final(xs1, seg_i, oth_i)
  h2 = run_final(xs2, seg_o, oth_o)

  w1 = jnp.tanh(h1 @ W_emb) @ a_emb
  w2 = jnp.tanh(h2 @ W_emb) @ a_emb
  beta = jax.nn.softmax(jnp.concatenate([w1, w2], axis=1), axis=1)
  xo = beta[:, 0:1] * h1 + beta[:, 1:2] * h2
  return (xo, out_rel)


# speedup vs baseline: 4.5338x; 1.5133x over previous
import functools

import jax
import jax.numpy as jnp
from jax import lax
from jax.experimental import pallas as pl
from jax.experimental.pallas import tpu as pltpu
from jax.experimental.pallas import tpu_sc as plsc

ALPHA = 0.2
N_NODES = 10000
NC = 2
NS = 16
L = 16
NW = NC * NS
CL1 = 32
CF = 16
WROW = 128
NACC = 10240
PADROW = 10008
RPT = NACC // NS
_CP = pltpu.CompilerParams(needs_layout_passes=False)
_MESH = dict(core_axis_name="c", subcore_axis_name="s")


def _zero_acc(sid, zsrc, acc_sh):
  zero = jnp.zeros((L,), jnp.float32)
  for r in range(16):
    for j in range(WROW // L):
      zsrc[r, pl.ds(j * L, L)] = zero

  def zloop(i, carry):
    pltpu.sync_copy(zsrc, acc_sh.at[pl.ds(sid * RPT + i * 16, 16)])
    return carry
  lax.fori_loop(0, RPT // 16, zloop, 0)


def _write_acc(cid, sid, acc_sh, obuf, out_hbm, rb):
  for b in range(RPT // rb):
    pltpu.sync_copy(acc_sh.at[pl.ds(sid * RPT + b * rb, rb)], obuf)
    pltpu.sync_copy(obuf, out_hbm.at[cid, pl.ds(sid * RPT + b * rb, rb)])


def _attn_e(tab1_v, tab2_v, sv, ov, srg):
  s1g = plsc.load_gather(tab1_v, [sv])
  s2g = plsc.load_gather(tab2_v, [ov])
  p = s1g + s2g + srg
  return jnp.exp(jnp.where(p > 0.0, -p, -ALPHA * p))


def _bcast(ee, k):
  return ee.at[jnp.full((L,), k, jnp.int32)].get(mode="promise_in_bounds")


def _layer1_pass(nch):
  C = CL1
  @functools.partial(
      pl.kernel,
      mesh=plsc.VectorSubcoreMesh(**_MESH),
      out_type=jax.ShapeDtypeStruct((NC, NACC, WROW), jnp.float32),
      compiler_params=_CP,
      scratch_types=[
          pltpu.VMEM((2, 4, C), jnp.int32),
          pltpu.VMEM((2, C, 16), jnp.float32),
          pltpu.VMEM((2, C, WROW), jnp.float32),
          pltpu.VMEM((2, C, WROW), jnp.float32),
          pltpu.VMEM((2, C), jnp.int32),
          pltpu.VMEM((NACC,), jnp.float32),
          pltpu.VMEM((NACC,), jnp.float32),
          pltpu.VMEM_SHARED((NACC, WROW), jnp.float32),
          pltpu.SemaphoreType.DMA,
          pltpu.SemaphoreType.DMA,
          pltpu.SemaphoreType.DMA,
          pltpu.SemaphoreType.DMA,
          pltpu.SemaphoreType.DMA,
          pltpu.SemaphoreType.DMA,
      ],
  )
  def kern(meta_hbm, rel_hbm, v_hbm, s1_hbm, s2_hbm, out_hbm,
           meta_v, rel_v, rows_v, scat_v, seg_idx_v, tab1_v, tab2_v,
           acc_sh, sm0, sm1, sg0, sg1, ss0, ss1):
    cid = lax.axis_index("c")
    sid = lax.axis_index("s")
    wid = cid * NS + sid
    sems_m, sems_g, sems_s = [sm0, sm1], [sg0, sg1], [ss0, ss1]

    pltpu.sync_copy(s1_hbm, tab1_v)
    pltpu.sync_copy(s2_hbm, tab2_v)
    _zero_acc(sid, rows_v.at[0, pl.ds(0, 16)], acc_sh)
    zero = jnp.zeros((L,), jnp.float32)
    for b in range(2):
      for r in range(C):
        scat_v[b, r, pl.ds(96, L)] = zero
        scat_v[b, r, pl.ds(112, L)] = zero
    plsc.subcore_barrier()

    lane = lax.iota(jnp.int32, 16)
    cbase = wid * nch

    def load_meta(cc, b):
      pltpu.async_copy(meta_hbm.at[cc], meta_v.at[b], sems_m[b])
      pltpu.async_copy(rel_hbm.at[cc], rel_v.at[b], sems_m[b])

    def wait_meta(b):
      pltpu.make_async_copy(meta_hbm.at[0], meta_v.at[b], sems_m[b]).wait()
      pltpu.make_async_copy(rel_hbm.at[0], rel_v.at[b], sems_m[b]).wait()

    def issue_gather(b):
      pltpu.async_copy(v_hbm.at[meta_v.at[b, 1]], rows_v.at[b], sems_g[b])

    def wait_gather(b):
      pltpu.make_async_copy(
          v_hbm.at[meta_v.at[b, 1]], rows_v.at[b], sems_g[b]).wait()

    load_meta(cbase, 0)
    wait_meta(0)
    issue_gather(0)
    load_meta(cbase + 1, 1)

    def outer(c2, carry):
      for b in range(2):
        c = c2 * 2 + b
        nb = 1 - b
        wait_meta(nb)
        issue_gather(nb)

        @pl.when(c2 > 0)
        def _():
          pltpu.make_async_copy(
              scat_v.at[b], acc_sh.at[seg_idx_v.at[b]], sems_s[b]).wait()

        wait_gather(b)
        for w in range(C // L):
          seg_idx_v[b, pl.ds(w * L, L)] = meta_v[b, 0, pl.ds(w * L, L)]
        for g in range(C // L):
          sv = meta_v[b, 0, pl.ds(g * L, L)]
          ov = meta_v[b, 1, pl.ds(g * L, L)]
          srg = plsc.bitcast(meta_v[b, 2, pl.ds(g * L, L)], jnp.float32)
          ee = _attn_e(tab1_v, tab2_v, sv, ov, srg)
          for k in range(L):
            r = g * L + k
            eb = _bcast(ee, k)
            for j in range(4):
              cs = pl.ds(j * L, L)
              scat_v[b, r, cs] = rows_v[b, r, cs] * eb
            scat_v[b, r, pl.ds(64, L)] = rel_v[b, r, :] * eb
            scat_v[b, r, pl.ds(80, L)] = jnp.where(lane == 0, eb, 0.0)
        pltpu.async_copy(scat_v.at[b], acc_sh.at[seg_idx_v.at[b]],
                         sems_s[b], add=True)
        cc = jnp.minimum(cbase + c + 2, cbase + nch - 1)
        load_meta(cc, b)
      return carry
    lax.fori_loop(0, nch // 2, outer, 0)

    wait_meta(1)
    wait_gather(0)
    for b in range(2):
      pltpu.make_async_copy(
          scat_v.at[b], acc_sh.at[seg_idx_v.at[b]], sems_s[b]).wait()
    plsc.subcore_barrier()
    _write_acc(cid, sid, acc_sh, scat_v.at[0], out_hbm, C)

  return kern


def _final_pass(nch):
  C = CF
  @functools.partial(
      pl.kernel,
      mesh=plsc.VectorSubcoreMesh(**_MESH),
      out_type=(jax.ShapeDtypeStruct((NC, NACC, WROW), jnp.float32),
                jax.ShapeDtypeStruct((NW, NACC), jnp.float32)),
      compiler_params=_CP,
      scratch_types=[
          pltpu.VMEM((2, 4, C), jnp.int32),
          pltpu.VMEM((2, C, WROW), jnp.float32),
          pltpu.VMEM((2, C, WROW), jnp.float32),
          pltpu.VMEM((2, C, WROW), jnp.float32),
          pltpu.VMEM((2, C), jnp.int32),
          pltpu.VMEM((NACC,), jnp.float32),
          pltpu.VMEM((NACC,), jnp.float32),
          pltpu.VMEM((NACC,), jnp.float32),
          pltpu.VMEM_SHARED((NACC, WROW), jnp.float32),
          pltpu.SemaphoreType.DMA,
          pltpu.SemaphoreType.DMA,
          pltpu.SemaphoreType.DMA,
          pltpu.SemaphoreType.DMA,
          pltpu.SemaphoreType.DMA,
          pltpu.SemaphoreType.DMA,
      ],
  )
  def kern(meta_hbm, relm_hbm, v_hbm, s1_hbm, s2_hbm, out_hbm, out2_hbm,
           meta_v, relm_v, rows_v, scat_v, seg_idx_v, tab1_v, tab2_v, rs_priv,
           acc_sh, sm0, sm1, sg0, sg1, ss0, ss1):
    cid = lax.axis_index("c")
    sid = lax.axis_index("s")
    wid = cid * NS + sid
    sems_m, sems_g, sems_s = [sm0, sm1], [sg0, sg1], [ss0, ss1]

    pltpu.sync_copy(s1_hbm, tab1_v)
    pltpu.sync_copy(s2_hbm, tab2_v)
    _zero_acc(sid, relm_v.at[0, pl.ds(0, 16)], acc_sh)
    zero = jnp.zeros((L,), jnp.float32)
    for i in range(NACC // L):
      rs_priv[pl.ds(i * L, L)] = zero
    plsc.subcore_barrier()

    cbase = wid * nch

    def load_meta(cc, b):
      pltpu.async_copy(meta_hbm.at[cc], meta_v.at[b], sems_m[b])
      pltpu.async_copy(relm_hbm.at[cc], relm_v.at[b], sems_m[b])

    def wait_meta(b):
      pltpu.make_async_copy(meta_hbm.at[0], meta_v.at[b], sems_m[b]).wait()
      pltpu.make_async_copy(relm_hbm.at[0], relm_v.at[b], sems_m[b]).wait()

    def issue_gather(b):
      pltpu.async_copy(v_hbm.at[meta_v.at[b, 1]], rows_v.at[b], sems_g[b])

    def wait_gather(b):
      pltpu.make_async_copy(
          v_hbm.at[meta_v.at[b, 1]], rows_v.at[b], sems_g[b]).wait()

    load_meta(cbase, 0)
    wait_meta(0)
    issue_gather(0)
    load_meta(cbase + 1, 1)

    def outer(c2, carry):
      for b in range(2):
        c = c2 * 2 + b
        nb = 1 - b
        wait_meta(nb)
        issue_gather(nb)

        @pl.when(c2 > 0)
        def _():
          pltpu.make_async_copy(
              scat_v.at[b], acc_sh.at[seg_idx_v.at[b]], sems_s[b]).wait()

        wait_gather(b)
        for w in range(C // L):
          seg_idx_v[b, pl.ds(w * L, L)] = meta_v[b, 0, pl.ds(w * L, L)]
        for g in range(C // L):
          sv = meta_v[b, 0, pl.ds(g * L, L)]
          ov = meta_v[b, 1, pl.ds(g * L, L)]
          srg = plsc.bitcast(meta_v[b, 2, pl.ds(g * L, L)], jnp.float32)
          ee = _attn_e(tab1_v, tab2_v, sv, ov, srg)
          plsc.addupdate_scatter(rs_priv, [sv], ee)
          for k in range(L):
            r = g * L + k
            eb = _bcast(ee, k)
            for j in range(WROW // L):
              cs = pl.ds(j * L, L)
              scat_v[b, r, cs] = (rows_v[b, r, cs] + relm_v[b, r, cs]) * eb
        pltpu.async_copy(scat_v.at[b], acc_sh.at[seg_idx_v.at[b]],
                         sems_s[b], add=True)
        cc = jnp.minimum(cbase + c + 2, cbase + nch - 1)
        load_meta(cc, b)
      return carry
    lax.fori_loop(0, nch // 2, outer, 0)

    wait_meta(1)
    wait_gather(0)
    for b in range(2):
      pltpu.make_async_copy(
          scat_v.at[b], acc_sh.at[seg_idx_v.at[b]], sems_s[b]).wait()
    plsc.subcore_barrier()
    _write_acc(cid, sid, acc_sh, scat_v.at[0], out_hbm, C)
    pltpu.sync_copy(rs_priv, out2_hbm.at[wid])

  return kern


def _f2i(x):
  return lax.bitcast_convert_type(x, jnp.int32)


def _pack_meta(seg, oth, sr, cw):
  n = seg.shape[0] // cw
  return jnp.concatenate([
      seg.reshape(n, 1, cw),
      oth.reshape(n, 1, cw),
      _f2i(sr).reshape(n, 1, cw),
      jnp.zeros((n, 1, cw), jnp.int32),
  ], axis=1)


def _pad_edges(seg, oth, ep):
  e = seg.shape[0]
  segp = jnp.full((ep,), PADROW, jnp.int32).at[:e].set(seg.astype(jnp.int32))
  othp = jnp.zeros((ep,), jnp.int32).at[:e].set(oth.astype(jnp.int32))
  return segp, othp


def _pad_rows(a, ep):
  return jnp.zeros((ep, a.shape[1]), jnp.float32).at[:a.shape[0]].set(a)


def _pad_tab(t):
  return jnp.zeros((NACC,), jnp.float32).at[:N_NODES].set(t)


def kernel(Corpus_, batch_inputs, entity_embeddings, relation_embed,
           edge_list, edge_type, edge_embed, edge_list_nhop, edge_type_nhop,
           a_in, a2_in, a_out, a2_out, W, a_final, a2_final, W_emb, a_emb):
  x = entity_embeddings
  R = relation_embed
  t0 = edge_type_nhop[:, 0]
  t1 = edge_type_nhop[:, 1]

  e_total = edge_list.shape[1] + edge_list_nhop.shape[1]
  epw = 64 * (-(-e_total // (NW * 64)))
  ep = NW * epw

  seg_in = jnp.concatenate([edge_list[0], edge_list_nhop[0]])
  oth_in = jnp.concatenate([edge_list[1], edge_list_nhop[1]])
  rel_l1 = _pad_rows(jnp.concatenate([edge_embed, R[t0] + R[t1]], axis=0), ep)
  rel_fin = _pad_rows(jnp.concatenate([R[edge_type], R[t0] + R[t1]], axis=0),
                      ep)

  seg_i, oth_i = _pad_edges(seg_in, oth_in, ep)
  seg_o, oth_o = _pad_edges(oth_in, seg_in, ep)
  rel_l1_s = rel_l1.reshape(ep // CL1, CL1, 16)

  l1_pass = _layer1_pass(epw // CL1)
  fin_pass = _final_pass(epw // CF)

  def run_layer1(a, a2, seg, oth):
    hs = []
    for h in range(2):
      a_s, a_d, a_r = a[h, :, :128], a[h, :, 128:256], a[h, :, 256:]
      v = a2[h, 0, :]
      s1 = _pad_tab(x @ (v @ a_s))
      s2 = _pad_tab(x @ (v @ a_d))
      sr = rel_l1 @ (v @ a_r)
      meta = _pack_meta(seg, oth, sr, CL1)
      v_tab = jnp.pad(x @ a_d.T, ((0, 0), (0, 64)))
      out = l1_pass(meta, rel_l1_s, v_tab, s1, s2)
      acc = (out[0] + out[1])[:N_NODES]
      rs = acc[:, 80]
      rs_cl = jnp.where(rs == 0.0, 1e-12, rs)
      num = (rs[:, None] * (x @ a_s.T) + acc[:, :64]
             + acc[:, 64:80] @ a_r.T)
      hs.append(num / rs_cl[:, None])
    return jax.nn.elu(jnp.concatenate(hs, axis=1))

  xs1 = run_layer1(a_in, a2_in, seg_i, oth_i)
  xs2 = run_layer1(a_out, a2_out, seg_o, oth_o)

  out_rel = R @ W

  a_sf, a_df, a_rf = a_final[:, :128], a_final[:, 128:256], a_final[:, 256:]
  vf = a2_final[0, :]
  wc = W @ a_rf.T
  relm_f = (rel_fin @ wc).reshape(ep // CF, CF, WROW)
  sr_f = rel_fin @ (W @ (vf @ a_rf))

  def run_final(xd, seg, oth):
    s1 = _pad_tab(xd @ (vf @ a_sf))
    s2 = _pad_tab(xd @ (vf @ a_df))
    meta = _pack_meta(seg, oth, sr_f, CF)
    v_tab = xd @ a_df.T
    out, rs_out = fin_pass(meta, relm_f, v_tab, s1, s2)
    acc = (out[0] + out[1])[:N_NODES]
    rs = jnp.sum(rs_out, axis=0)[:N_NODES]
    rs_cl = jnp.where(rs == 0.0, 1e-12, rs)
    num = rs[:, None] * (xd @ a_sf.T) + acc
    return jax.nn.elu(num / rs_cl[:, None])

  h1 = run_final(xs1, seg_i, oth_i)
  h2 = run_final(xs2, seg_o, oth_o)

  w1 = jnp.tanh(h1 @ W_emb) @ a_emb
  w2 = jnp.tanh(h2 @ W_emb) @ a_emb
  beta = jax.nn.softmax(jnp.concatenate([w1, w2], axis=1), axis=1)
  xo = beta[:, 0:1] * h1 + beta[:, 1:2] * h2
  return (xo, out_rel)
